# Initial kernel scaffold; baseline (speedup 1.0000x reference)
import jax
import jax.numpy as jnp
from jax.experimental import pallas as pl


def kernel(x, pos, edge_idx, mask, W_e, b_e, W_n, b_n, ln_scale, ln_bias):
    raise NotImplementedError("write your pallas kernel here")


# speedup vs baseline: 928.4271x; 928.4271x over previous
import functools

import jax
import jax.numpy as jnp
from jax import lax
from jax.experimental import pallas as pl
from jax.experimental.pallas import tpu as pltpu
from jax.experimental.pallas import tpu_sc as plsc

N = 10000
K = 32
D = 128
O = 128

NC = 2
NS = 16
NW = NC * NS
L = 16
N_PAD = 10240
NPT = N_PAD // NW
CH = 4
RG = CH * K
NCHUNK = NPT // CH
RB = 1024


def _dense_pre_body(x_ref, pos_ref, wcat_ref, wep_ref, be_ref,
                    base_ref, c_ref, xw1_ref):
    y = jnp.dot(x_ref[...], wcat_ref[...], preferred_element_type=jnp.float32)
    p = (pos_ref[:, 0:1] * wep_ref[0:1, :] + pos_ref[:, 1:2] * wep_ref[1:2, :])
    base_ref[...] = y[:, :O] - p + be_ref[...]
    c_ref[...] = y[:, O:2 * O] + p
    xw1_ref[...] = y[:, 2 * O:]


def _node_body(xw1_ref, agg_ref, wn2_ref, bn_ref, lns_ref, lnb_ref, mask_ref,
               out_ref):
    h = (xw1_ref[...]
         + jnp.dot(agg_ref[...], wn2_ref[...], preferred_element_type=jnp.float32)
         + bn_ref[...])
    mu = jnp.mean(h, axis=1, keepdims=True)
    hc = h - mu
    var = jnp.mean(hc * hc, axis=1, keepdims=True)
    hn = hc * lax.rsqrt(var + 1e-5) * lns_ref[...] + lnb_ref[...]
    out_ref[...] = jnp.maximum(hn, 0.0) * mask_ref[...]


def _sc_gather_body(base_hbm, c_hbm, idx_hbm, out_hbm,
                    idx_v, base_v, rows0, rows1, out_v, sem0, sem1):
    cid = lax.axis_index("c")
    sid = lax.axis_index("s")
    wid = sid * NC + cid
    node0 = wid * NPT
    chunk0 = wid * NCHUNK

    pltpu.sync_copy(idx_hbm.at[pl.ds(chunk0, NCHUNK)], idx_v)
    pltpu.sync_copy(base_hbm.at[pl.ds(node0, NPT)], base_v)

    rows = (rows0, rows1)
    sems = (sem0, sem1)

    pltpu.async_copy(c_hbm.at[idx_v.at[0]], rows0, sem0)

    inv_k = jnp.float32(1.0 / K)

    def pair_body(gp, carry):
        for b in range(2):
            g = gp * 2 + b
            @pl.when(g + 1 < NCHUNK)
            def _():
                pltpu.async_copy(c_hbm.at[idx_v.at[g + 1]], rows[1 - b],
                                 sems[1 - b])
            pltpu.make_async_copy(c_hbm.at[idx_v.at[g]], rows[b],
                                  sems[b]).wait()
            rb = rows[b]
            for i in range(CH):
                nrow = g * CH + i
                bjs = [base_v[nrow, pl.ds(L * j, L)] for j in range(8)]

                def k_body(k, accs):
                    r = i * K + k
                    return tuple(
                        accs[j] + jnp.maximum(rb[r, pl.ds(L * j, L)] + bjs[j],
                                              0.0)
                        for j in range(8))

                accs = lax.fori_loop(
                    0, K, k_body,
                    tuple(jnp.zeros((L,), jnp.float32) for _ in range(8)))
                for j in range(8):
                    out_v[nrow, pl.ds(L * j, L)] = accs[j] * inv_k
        return carry

    lax.fori_loop(0, NCHUNK // 2, pair_body, jnp.int32(0))

    pltpu.sync_copy(out_v, out_hbm.at[pl.ds(node0, NPT)])


_sc_gather = functools.partial(
    pl.kernel,
    out_type=jax.ShapeDtypeStruct((N_PAD, O), jnp.float32),
    mesh=plsc.VectorSubcoreMesh(core_axis_name="c", subcore_axis_name="s",
                                num_cores=NC, num_subcores=NS),
    scratch_types=[
        pltpu.VMEM((NW * NCHUNK // NW, RG), jnp.int32),
        pltpu.VMEM((NPT, O), jnp.float32),
        pltpu.VMEM((RG, O), jnp.float32),
        pltpu.VMEM((RG, O), jnp.float32),
        pltpu.VMEM((NPT, O), jnp.float32),
        pltpu.SemaphoreType.DMA,
        pltpu.SemaphoreType.DMA,
    ],
)(_sc_gather_body)


def kernel(x, pos, edge_idx, mask, W_e, b_e, W_n, b_n, ln_scale, ln_bias):
    B = x.shape[0]
    x2 = x.reshape(B * N, D)
    pos2 = pos.reshape(B * N, 2)
    pad = N_PAD - N
    x_pad = jnp.pad(x2, ((0, pad), (0, 0)))
    pos_pad = jnp.pad(pos2, ((0, pad), (0, 0)))
    idx_pad = jnp.pad(edge_idx.reshape(B * N, K).astype(jnp.int32),
                      ((0, pad), (0, 0))).reshape(NW * NCHUNK, RG)
    mask_pad = jnp.pad(mask.reshape(B * N, 1), ((0, pad), (0, 0)))

    wcat = jnp.concatenate([W_e[:D], W_e[D:2 * D], W_n[:D]], axis=1)
    wep = W_e[2 * D:]
    wn2 = W_n[D:]

    grid = N_PAD // RB
    base_c_xw1 = pl.pallas_call(
        _dense_pre_body,
        grid=(grid,),
        in_specs=[
            pl.BlockSpec((RB, D), lambda i: (i, 0)),
            pl.BlockSpec((RB, 2), lambda i: (i, 0)),
            pl.BlockSpec((D, 3 * O), lambda i: (0, 0)),
            pl.BlockSpec((2, O), lambda i: (0, 0)),
            pl.BlockSpec((1, O), lambda i: (0, 0)),
        ],
        out_specs=[
            pl.BlockSpec((RB, O), lambda i: (i, 0)),
            pl.BlockSpec((RB, O), lambda i: (i, 0)),
            pl.BlockSpec((RB, O), lambda i: (i, 0)),
        ],
        out_shape=[
            jax.ShapeDtypeStruct((N_PAD, O), jnp.float32),
            jax.ShapeDtypeStruct((N_PAD, O), jnp.float32),
            jax.ShapeDtypeStruct((N_PAD, O), jnp.float32),
        ],
    )(x_pad, pos_pad, wcat, wep, b_e.reshape(1, O))
    base, c, xw1 = base_c_xw1

    agg = _sc_gather(base, c, idx_pad)

    out_pad = pl.pallas_call(
        _node_body,
        grid=(grid,),
        in_specs=[
            pl.BlockSpec((RB, O), lambda i: (i, 0)),
            pl.BlockSpec((RB, O), lambda i: (i, 0)),
            pl.BlockSpec((O, O), lambda i: (0, 0)),
            pl.BlockSpec((1, O), lambda i: (0, 0)),
            pl.BlockSpec((1, O), lambda i: (0, 0)),
            pl.BlockSpec((1, O), lambda i: (0, 0)),
            pl.BlockSpec((RB, 1), lambda i: (i, 0)),
        ],
        out_specs=pl.BlockSpec((RB, O), lambda i: (i, 0)),
        out_shape=jax.ShapeDtypeStruct((N_PAD, O), jnp.float32),
    )(xw1, agg, wn2, b_n.reshape(1, O), ln_scale.reshape(1, O),
      ln_bias.reshape(1, O), mask_pad)

    return out_pad[:N].reshape(B, N, O)


# speedup vs baseline: 933.2876x; 1.0052x over previous
import functools

import jax
import jax.numpy as jnp
from jax import lax
from jax.experimental import pallas as pl
from jax.experimental.pallas import tpu as pltpu
from jax.experimental.pallas import tpu_sc as plsc

N = 10000
K = 32
D = 128
O = 128

NC = 2
NS = 16
NW = NC * NS
L = 16
N_PAD = 10240
NPT = N_PAD // NW
CH = 2
RG = CH * K
NCHUNK = NPT // CH
NB = 4
KU = 8
RB = 1024


def _dense_pre_body(x_ref, pos_ref, wcat_ref, wep_ref, be_ref,
                    base_ref, c_ref, xw1_ref):
    y = jnp.dot(x_ref[...], wcat_ref[...], preferred_element_type=jnp.float32)
    p = (pos_ref[:, 0:1] * wep_ref[0:1, :] + pos_ref[:, 1:2] * wep_ref[1:2, :])
    base_ref[...] = y[:, :O] - p + be_ref[...]
    c_ref[...] = y[:, O:2 * O] + p
    xw1_ref[...] = y[:, 2 * O:]


def _node_body(xw1_ref, agg_ref, wn2_ref, bn_ref, lns_ref, lnb_ref, mask_ref,
               out_ref):
    h = (xw1_ref[...]
         + jnp.dot(agg_ref[...], wn2_ref[...], preferred_element_type=jnp.float32)
         + bn_ref[...])
    mu = jnp.mean(h, axis=1, keepdims=True)
    hc = h - mu
    var = jnp.mean(hc * hc, axis=1, keepdims=True)
    hn = hc * lax.rsqrt(var + 1e-5) * lns_ref[...] + lnb_ref[...]
    out_ref[...] = jnp.maximum(hn, 0.0) * mask_ref[...]


def _sc_gather_body(base_hbm, c_hbm, idx_hbm, out_hbm,
                    idx_v, base_v, rows0, rows1, rows2, rows3,
                    sem0, sem1, sem2, sem3):
    cid = lax.axis_index("c")
    sid = lax.axis_index("s")
    wid = sid * NC + cid
    node0 = wid * NPT
    chunk0 = wid * NCHUNK

    pltpu.sync_copy(idx_hbm.at[pl.ds(chunk0, NCHUNK)], idx_v)
    pltpu.sync_copy(base_hbm.at[pl.ds(node0, NPT)], base_v)

    rows = (rows0, rows1, rows2, rows3)
    sems = (sem0, sem1, sem2, sem3)

    for b in range(NB):
        pltpu.async_copy(c_hbm.at[idx_v.at[b]], rows[b], sems[b])

    inv_k = jnp.float32(1.0 / K)

    def ring_body(it, carry):
        for b in range(NB):
            g = it * NB + b
            pltpu.make_async_copy(c_hbm.at[idx_v.at[g]], rows[b],
                                  sems[b]).wait()
            rb = rows[b]
            for i in range(CH):
                nrow = g * CH + i
                bjs = [base_v[nrow, pl.ds(L * j, L)] for j in range(8)]

                def kg_body(kg, accs):
                    r0 = i * K + kg * KU
                    for kk in range(KU):
                        accs = tuple(
                            accs[j] + jnp.maximum(
                                rb[r0 + kk, pl.ds(L * j, L)] + bjs[j], 0.0)
                            for j in range(8))
                    return accs

                accs = lax.fori_loop(
                    0, K // KU, kg_body,
                    tuple(jnp.zeros((L,), jnp.float32) for _ in range(8)))
                for j in range(8):
                    base_v[nrow, pl.ds(L * j, L)] = accs[j] * inv_k
            @pl.when(g + NB < NCHUNK)
            def _():
                pltpu.async_copy(c_hbm.at[idx_v.at[g + NB]], rows[b], sems[b])
        return carry

    lax.fori_loop(0, NCHUNK // NB, ring_body, jnp.int32(0))

    pltpu.sync_copy(base_v, out_hbm.at[pl.ds(node0, NPT)])


_sc_gather = functools.partial(
    pl.kernel,
    out_type=jax.ShapeDtypeStruct((N_PAD, O), jnp.float32),
    mesh=plsc.VectorSubcoreMesh(core_axis_name="c", subcore_axis_name="s",
                                num_cores=NC, num_subcores=NS),
    scratch_types=[
        pltpu.VMEM((NCHUNK, RG), jnp.int32),
        pltpu.VMEM((NPT, O), jnp.float32),
        pltpu.VMEM((RG, O), jnp.float32),
        pltpu.VMEM((RG, O), jnp.float32),
        pltpu.VMEM((RG, O), jnp.float32),
        pltpu.VMEM((RG, O), jnp.float32),
        pltpu.SemaphoreType.DMA,
        pltpu.SemaphoreType.DMA,
        pltpu.SemaphoreType.DMA,
        pltpu.SemaphoreType.DMA,
    ],
)(_sc_gather_body)


def kernel(x, pos, edge_idx, mask, W_e, b_e, W_n, b_n, ln_scale, ln_bias):
    B = x.shape[0]
    x2 = x.reshape(B * N, D)
    pos2 = pos.reshape(B * N, 2)
    pad = N_PAD - N
    x_pad = jnp.pad(x2, ((0, pad), (0, 0)))
    pos_pad = jnp.pad(pos2, ((0, pad), (0, 0)))
    idx_pad = jnp.pad(edge_idx.reshape(B * N, K).astype(jnp.int32),
                      ((0, pad), (0, 0))).reshape(NW * NCHUNK, RG)
    mask_pad = jnp.pad(mask.reshape(B * N, 1), ((0, pad), (0, 0)))

    wcat = jnp.concatenate([W_e[:D], W_e[D:2 * D], W_n[:D]], axis=1)
    wep = W_e[2 * D:]
    wn2 = W_n[D:]

    grid = N_PAD // RB
    base_c_xw1 = pl.pallas_call(
        _dense_pre_body,
        grid=(grid,),
        in_specs=[
            pl.BlockSpec((RB, D), lambda i: (i, 0)),
            pl.BlockSpec((RB, 2), lambda i: (i, 0)),
            pl.BlockSpec((D, 3 * O), lambda i: (0, 0)),
            pl.BlockSpec((2, O), lambda i: (0, 0)),
            pl.BlockSpec((1, O), lambda i: (0, 0)),
        ],
        out_specs=[
            pl.BlockSpec((RB, O), lambda i: (i, 0)),
            pl.BlockSpec((RB, O), lambda i: (i, 0)),
            pl.BlockSpec((RB, O), lambda i: (i, 0)),
        ],
        out_shape=[
            jax.ShapeDtypeStruct((N_PAD, O), jnp.float32),
            jax.ShapeDtypeStruct((N_PAD, O), jnp.float32),
            jax.ShapeDtypeStruct((N_PAD, O), jnp.float32),
        ],
    )(x_pad, pos_pad, wcat, wep, b_e.reshape(1, O))
    base, c, xw1 = base_c_xw1

    agg = _sc_gather(base, c, idx_pad)

    out_pad = pl.pallas_call(
        _node_body,
        grid=(grid,),
        in_specs=[
            pl.BlockSpec((RB, O), lambda i: (i, 0)),
            pl.BlockSpec((RB, O), lambda i: (i, 0)),
            pl.BlockSpec((O, O), lambda i: (0, 0)),
            pl.BlockSpec((1, O), lambda i: (0, 0)),
            pl.BlockSpec((1, O), lambda i: (0, 0)),
            pl.BlockSpec((1, O), lambda i: (0, 0)),
            pl.BlockSpec((RB, 1), lambda i: (i, 0)),
        ],
        out_specs=pl.BlockSpec((RB, O), lambda i: (i, 0)),
        out_shape=jax.ShapeDtypeStruct((N_PAD, O), jnp.float32),
    )(xw1, agg, wn2, b_n.reshape(1, O), ln_scale.reshape(1, O),
      ln_bias.reshape(1, O), mask_pad)

    return out_pad[:N].reshape(B, N, O)


# speedup vs baseline: 1260.4326x; 1.3505x over previous
import functools

import numpy as np

import jax
import jax.numpy as jnp
from jax import lax
from jax.experimental import pallas as pl
from jax.experimental.pallas import tpu as pltpu
from jax.experimental.pallas import tpu_sc as plsc

N = 10000
K = 32
D = 128
O = 128

NC = 2
NS = 16
NW = NC * NS
L = 16
N_PAD = 10240
NPT = N_PAD // NW
CH = 2
RG = CH * K
NCHUNK = NPT // CH
NB = 4
KU = 8

_PERM = np.concatenate([
    np.concatenate([np.arange(g * 32, (g + 1) * 32, 2),
                    np.arange(g * 32 + 1, (g + 1) * 32, 2)])
    for g in range(4)])
RB = 1024


def _dense_pre_body(x_ref, pos_ref, wcat_ref, wep_ref, be_ref,
                    base_ref, c_ref, xw1_ref):
    y = jnp.dot(x_ref[...], wcat_ref[...], preferred_element_type=jnp.float32)
    p2 = (pos_ref[:, 0:1] * wep_ref[0:1, :] + pos_ref[:, 1:2] * wep_ref[1:2, :])
    base_ref[...] = y[:, :O] - p2[:, :O] + be_ref[...]
    c_ref[...] = (y[:, O:2 * O] + p2[:, O:]).astype(jnp.bfloat16)
    xw1_ref[...] = y[:, 2 * O:]


def _node_body(xw1_ref, agg_ref, wn2_ref, bn_ref, lns_ref, lnb_ref, mask_ref,
               out_ref):
    h = (xw1_ref[...]
         + jnp.dot(agg_ref[...], wn2_ref[...], preferred_element_type=jnp.float32)
         + bn_ref[...])
    mu = jnp.mean(h, axis=1, keepdims=True)
    hc = h - mu
    var = jnp.mean(hc * hc, axis=1, keepdims=True)
    hn = hc * lax.rsqrt(var + 1e-5) * lns_ref[...] + lnb_ref[...]
    out_ref[...] = jnp.maximum(hn, 0.0) * mask_ref[...]


def _sc_gather_body(base_hbm, c_hbm, idx_hbm, out_hbm,
                    idx_v, base_v, rows0, rows1, rows2, rows3,
                    sem0, sem1, sem2, sem3):
    cid = lax.axis_index("c")
    sid = lax.axis_index("s")
    wid = sid * NC + cid
    node0 = wid * NPT
    chunk0 = wid * NCHUNK

    pltpu.sync_copy(idx_hbm.at[pl.ds(chunk0, NCHUNK)], idx_v)
    pltpu.sync_copy(base_hbm.at[pl.ds(node0, NPT)], base_v)

    rows = (rows0, rows1, rows2, rows3)
    sems = (sem0, sem1, sem2, sem3)

    for b in range(NB):
        pltpu.async_copy(c_hbm.at[idx_v.at[b]], rows[b], sems[b])

    inv_k = jnp.float32(1.0 / K)

    def ring_body(it, carry):
        for b in range(NB):
            g = it * NB + b
            pltpu.make_async_copy(c_hbm.at[idx_v.at[g]], rows[b],
                                  sems[b]).wait()
            rb = rows[b]
            for i in range(CH):
                nrow = g * CH + i
                bjs = [base_v[nrow, pl.ds(L * j, L)] for j in range(8)]

                def kg_body(kg, accs):
                    r0 = i * K + kg * KU
                    new = list(accs)
                    hi_mask = jnp.full((L,), -65536, jnp.int32)
                    for kk in range(KU):
                        for jj in range(4):
                            vi = rb[r0 + kk, pl.ds(L * jj, L)]
                            ev = lax.bitcast_convert_type(vi << 16,
                                                          jnp.float32)
                            od = lax.bitcast_convert_type(vi & hi_mask,
                                                          jnp.float32)
                            new[2 * jj] = new[2 * jj] + jnp.maximum(
                                ev + bjs[2 * jj], 0.0)
                            new[2 * jj + 1] = new[2 * jj + 1] + jnp.maximum(
                                od + bjs[2 * jj + 1], 0.0)
                    return tuple(new)

                accs = lax.fori_loop(
                    0, K // KU, kg_body,
                    tuple(jnp.zeros((L,), jnp.float32) for _ in range(8)))
                for j in range(8):
                    base_v[nrow, pl.ds(L * j, L)] = accs[j] * inv_k
            @pl.when(g + NB < NCHUNK)
            def _():
                pltpu.async_copy(c_hbm.at[idx_v.at[g + NB]], rows[b], sems[b])
        return carry

    lax.fori_loop(0, NCHUNK // NB, ring_body, jnp.int32(0))

    pltpu.sync_copy(base_v, out_hbm.at[pl.ds(node0, NPT)])


_sc_gather = functools.partial(
    pl.kernel,
    out_type=jax.ShapeDtypeStruct((N_PAD, O), jnp.float32),
    mesh=plsc.VectorSubcoreMesh(core_axis_name="c", subcore_axis_name="s",
                                num_cores=NC, num_subcores=NS),
    compiler_params=pltpu.CompilerParams(use_tc_tiling_on_sc=False),
    scratch_types=[
        pltpu.VMEM((NCHUNK, RG), jnp.int32),
        pltpu.VMEM((NPT, O), jnp.float32),
        pltpu.VMEM((RG, O // 2), jnp.int32),
        pltpu.VMEM((RG, O // 2), jnp.int32),
        pltpu.VMEM((RG, O // 2), jnp.int32),
        pltpu.VMEM((RG, O // 2), jnp.int32),
        pltpu.SemaphoreType.DMA,
        pltpu.SemaphoreType.DMA,
        pltpu.SemaphoreType.DMA,
        pltpu.SemaphoreType.DMA,
    ],
)(_sc_gather_body)


def kernel(x, pos, edge_idx, mask, W_e, b_e, W_n, b_n, ln_scale, ln_bias):
    B = x.shape[0]
    x2 = x.reshape(B * N, D)
    pos2 = pos.reshape(B * N, 2)
    pad = N_PAD - N
    x_pad = jnp.pad(x2, ((0, pad), (0, 0)))
    pos_pad = jnp.pad(pos2, ((0, pad), (0, 0)))
    idx_pad = jnp.pad(edge_idx.reshape(B * N, K).astype(jnp.int32),
                      ((0, pad), (0, 0))).reshape(NW * NCHUNK, RG)
    mask_pad = jnp.pad(mask.reshape(B * N, 1), ((0, pad), (0, 0)))

    perm = jnp.asarray(_PERM)
    wcat = jnp.concatenate([W_e[:D][:, perm], W_e[D:2 * D], W_n[:D]], axis=1)
    wep = jnp.concatenate([W_e[2 * D:][:, perm], W_e[2 * D:]], axis=1)
    wn2 = W_n[D:][perm, :]

    grid = N_PAD // RB
    base_c_xw1 = pl.pallas_call(
        _dense_pre_body,
        grid=(grid,),
        in_specs=[
            pl.BlockSpec((RB, D), lambda i: (i, 0)),
            pl.BlockSpec((RB, 2), lambda i: (i, 0)),
            pl.BlockSpec((D, 3 * O), lambda i: (0, 0)),
            pl.BlockSpec((2, 2 * O), lambda i: (0, 0)),
            pl.BlockSpec((1, O), lambda i: (0, 0)),
        ],
        out_specs=[
            pl.BlockSpec((RB, O), lambda i: (i, 0)),
            pl.BlockSpec((RB, O), lambda i: (i, 0)),
            pl.BlockSpec((RB, O), lambda i: (i, 0)),
        ],
        out_shape=[
            jax.ShapeDtypeStruct((N_PAD, O), jnp.float32),
            jax.ShapeDtypeStruct((N_PAD, O), jnp.bfloat16),
            jax.ShapeDtypeStruct((N_PAD, O), jnp.float32),
        ],
    )(x_pad, pos_pad, wcat, wep, b_e[perm].reshape(1, O))
    base, c_bf, xw1 = base_c_xw1
    c_i32 = lax.bitcast_convert_type(c_bf.reshape(N_PAD, O // 2, 2), jnp.int32)

    agg = _sc_gather(base, c_i32, idx_pad)

    out_pad = pl.pallas_call(
        _node_body,
        grid=(grid,),
        in_specs=[
            pl.BlockSpec((RB, O), lambda i: (i, 0)),
            pl.BlockSpec((RB, O), lambda i: (i, 0)),
            pl.BlockSpec((O, O), lambda i: (0, 0)),
            pl.BlockSpec((1, O), lambda i: (0, 0)),
            pl.BlockSpec((1, O), lambda i: (0, 0)),
            pl.BlockSpec((1, O), lambda i: (0, 0)),
            pl.BlockSpec((RB, 1), lambda i: (i, 0)),
        ],
        out_specs=pl.BlockSpec((RB, O), lambda i: (i, 0)),
        out_shape=jax.ShapeDtypeStruct((N_PAD, O), jnp.float32),
    )(xw1, agg, wn2, b_n.reshape(1, O), ln_scale.reshape(1, O),
      ln_bias.reshape(1, O), mask_pad)

    return out_pad[:N].reshape(B, N, O)


# speedup vs baseline: 2015.7108x; 1.5992x over previous
import functools

import numpy as np

import jax
import jax.numpy as jnp
from jax import lax
from jax.experimental import pallas as pl
from jax.experimental.pallas import tpu as pltpu
from jax.experimental.pallas import tpu_sc as plsc

N = 10000
K = 32
D = 128
O = 128

NC = 2
NS = 16
NW = NC * NS
L = 16
N_PAD = 10240
NPT = N_PAD // NW
CH = 2
RG = CH * K
NCHUNK = NPT // CH
NB = 4
KU = 8

_PERM = np.concatenate([
    np.concatenate([np.arange(g * 32, (g + 1) * 32, 2),
                    np.arange(g * 32 + 1, (g + 1) * 32, 2)])
    for g in range(4)])
RB = 1024


def _dense_pre_body(x_ref, pos_ref, wcat_ref, wep_ref, be_ref,
                    base_ref, c_ref, xw1_ref):
    y = jnp.dot(x_ref[...], wcat_ref[...], preferred_element_type=jnp.float32)
    p2 = (pos_ref[:, 0:1] * wep_ref[0:1, :] + pos_ref[:, 1:2] * wep_ref[1:2, :])
    base_ref[...] = y[:, :O] - p2[:, :O] + be_ref[...]
    c_ref[...] = (y[:, O:2 * O] + p2[:, O:]).astype(jnp.bfloat16)
    xw1_ref[...] = y[:, 2 * O:]


def _node_body(xw1_ref, agg_ref, wn2_ref, bn_ref, lns_ref, lnb_ref, mask_ref,
               out_ref):
    h = (xw1_ref[...]
         + jnp.dot(agg_ref[...], wn2_ref[...], preferred_element_type=jnp.float32)
         + bn_ref[...])
    mu = jnp.mean(h, axis=1, keepdims=True)
    hc = h - mu
    var = jnp.mean(hc * hc, axis=1, keepdims=True)
    hn = hc * lax.rsqrt(var + 1e-5) * lns_ref[...] + lnb_ref[...]
    out_ref[...] = jnp.maximum(hn, 0.0) * mask_ref[...]


def _sc_gather_body(base_hbm, c_hbm, idx_hbm, out_hbm,
                    c_sp, idx_v, base_v, rows0, rows1, rows2, rows3,
                    sem0, sem1, sem2, sem3):
    cid = lax.axis_index("c")
    sid = lax.axis_index("s")
    wid = sid * NC + cid
    node0 = wid * NPT
    chunk0 = wid * NCHUNK

    rps = N_PAD // NS
    pltpu.sync_copy(c_hbm.at[pl.ds(sid * rps, rps)],
                    c_sp.at[pl.ds(sid * rps, rps)])

    pltpu.sync_copy(idx_hbm.at[pl.ds(chunk0, NCHUNK)], idx_v)
    pltpu.sync_copy(base_hbm.at[pl.ds(node0, NPT)], base_v)

    rows = (rows0, rows1, rows2, rows3)
    sems = (sem0, sem1, sem2, sem3)

    plsc.subcore_barrier()

    for b in range(NB):
        pltpu.async_copy(c_sp.at[idx_v.at[b]], rows[b], sems[b])

    inv_k = jnp.float32(1.0 / K)

    def ring_body(it, carry):
        for b in range(NB):
            g = it * NB + b
            pltpu.make_async_copy(c_sp.at[idx_v.at[g]], rows[b],
                                  sems[b]).wait()
            rb = rows[b]
            for i in range(CH):
                nrow = g * CH + i
                bjs = [base_v[nrow, pl.ds(L * j, L)] for j in range(8)]

                def kg_body(kg, accs):
                    r0 = i * K + kg * KU
                    new = list(accs)
                    hi_mask = jnp.full((L,), -65536, jnp.int32)
                    for kk in range(KU):
                        for jj in range(4):
                            vi = rb[r0 + kk, pl.ds(L * jj, L)]
                            ev = lax.bitcast_convert_type(vi << 16,
                                                          jnp.float32)
                            od = lax.bitcast_convert_type(vi & hi_mask,
                                                          jnp.float32)
                            new[2 * jj] = new[2 * jj] + jnp.maximum(
                                ev + bjs[2 * jj], 0.0)
                            new[2 * jj + 1] = new[2 * jj + 1] + jnp.maximum(
                                od + bjs[2 * jj + 1], 0.0)
                    return tuple(new)

                accs = lax.fori_loop(
                    0, K // KU, kg_body,
                    tuple(jnp.zeros((L,), jnp.float32) for _ in range(8)))
                for j in range(8):
                    base_v[nrow, pl.ds(L * j, L)] = accs[j] * inv_k
            @pl.when(g + NB < NCHUNK)
            def _():
                pltpu.async_copy(c_sp.at[idx_v.at[g + NB]], rows[b], sems[b])
        return carry

    lax.fori_loop(0, NCHUNK // NB, ring_body, jnp.int32(0))

    pltpu.sync_copy(base_v, out_hbm.at[pl.ds(node0, NPT)])


_sc_gather = functools.partial(
    pl.kernel,
    out_type=jax.ShapeDtypeStruct((N_PAD, O), jnp.float32),
    mesh=plsc.VectorSubcoreMesh(core_axis_name="c", subcore_axis_name="s",
                                num_cores=NC, num_subcores=NS),
    compiler_params=pltpu.CompilerParams(use_tc_tiling_on_sc=False),
    scratch_types=[
        pltpu.VMEM_SHARED((N_PAD, O // 2), jnp.int32),
        pltpu.VMEM((NCHUNK, RG), jnp.int32),
        pltpu.VMEM((NPT, O), jnp.float32),
        pltpu.VMEM((RG, O // 2), jnp.int32),
        pltpu.VMEM((RG, O // 2), jnp.int32),
        pltpu.VMEM((RG, O // 2), jnp.int32),
        pltpu.VMEM((RG, O // 2), jnp.int32),
        pltpu.SemaphoreType.DMA,
        pltpu.SemaphoreType.DMA,
        pltpu.SemaphoreType.DMA,
        pltpu.SemaphoreType.DMA,
    ],
)(_sc_gather_body)


def kernel(x, pos, edge_idx, mask, W_e, b_e, W_n, b_n, ln_scale, ln_bias):
    B = x.shape[0]
    x2 = x.reshape(B * N, D)
    pos2 = pos.reshape(B * N, 2)
    pad = N_PAD - N
    x_pad = jnp.pad(x2, ((0, pad), (0, 0)))
    pos_pad = jnp.pad(pos2, ((0, pad), (0, 0)))
    idx_pad = jnp.pad(edge_idx.reshape(B * N, K).astype(jnp.int32),
                      ((0, pad), (0, 0))).reshape(NW * NCHUNK, RG)
    mask_pad = jnp.pad(mask.reshape(B * N, 1), ((0, pad), (0, 0)))

    perm = jnp.asarray(_PERM)
    wcat = jnp.concatenate([W_e[:D][:, perm], W_e[D:2 * D], W_n[:D]], axis=1)
    wep = jnp.concatenate([W_e[2 * D:][:, perm], W_e[2 * D:]], axis=1)
    wn2 = W_n[D:][perm, :]

    grid = N_PAD // RB
    base_c_xw1 = pl.pallas_call(
        _dense_pre_body,
        grid=(grid,),
        in_specs=[
            pl.BlockSpec((RB, D), lambda i: (i, 0)),
            pl.BlockSpec((RB, 2), lambda i: (i, 0)),
            pl.BlockSpec((D, 3 * O), lambda i: (0, 0)),
            pl.BlockSpec((2, 2 * O), lambda i: (0, 0)),
            pl.BlockSpec((1, O), lambda i: (0, 0)),
        ],
        out_specs=[
            pl.BlockSpec((RB, O), lambda i: (i, 0)),
            pl.BlockSpec((RB, O), lambda i: (i, 0)),
            pl.BlockSpec((RB, O), lambda i: (i, 0)),
        ],
        out_shape=[
            jax.ShapeDtypeStruct((N_PAD, O), jnp.float32),
            jax.ShapeDtypeStruct((N_PAD, O), jnp.bfloat16),
            jax.ShapeDtypeStruct((N_PAD, O), jnp.float32),
        ],
    )(x_pad, pos_pad, wcat, wep, b_e[perm].reshape(1, O))
    base, c_bf, xw1 = base_c_xw1
    c_i32 = lax.bitcast_convert_type(c_bf.reshape(N_PAD, O // 2, 2), jnp.int32)

    agg = _sc_gather(base, c_i32, idx_pad)

    out_pad = pl.pallas_call(
        _node_body,
        grid=(grid,),
        in_specs=[
            pl.BlockSpec((RB, O), lambda i: (i, 0)),
            pl.BlockSpec((RB, O), lambda i: (i, 0)),
            pl.BlockSpec((O, O), lambda i: (0, 0)),
            pl.BlockSpec((1, O), lambda i: (0, 0)),
            pl.BlockSpec((1, O), lambda i: (0, 0)),
            pl.BlockSpec((1, O), lambda i: (0, 0)),
            pl.BlockSpec((RB, 1), lambda i: (i, 0)),
        ],
        out_specs=pl.BlockSpec((RB, O), lambda i: (i, 0)),
        out_shape=jax.ShapeDtypeStruct((N_PAD, O), jnp.float32),
    )(xw1, agg, wn2, b_n.reshape(1, O), ln_scale.reshape(1, O),
      ln_bias.reshape(1, O), mask_pad)

    return out_pad[:N].reshape(B, N, O)


# speedup vs baseline: 2372.3834x; 1.1769x over previous
import functools

import numpy as np

import jax
import jax.numpy as jnp
from jax import lax
from jax.experimental import pallas as pl
from jax.experimental.pallas import tpu as pltpu
from jax.experimental.pallas import tpu_sc as plsc

N = 10000
K = 32
D = 128
O = 128

NC = 2
NS = 16
NW = NC * NS
L = 16
N_PAD = 10240
NPT = N_PAD // NW
CH = 2
RG = CH * K
NCHUNK = NPT // CH
NB = 4
KU = 8

_PERM = np.concatenate([
    np.concatenate([np.arange(16 * g, 16 * g + 16),
                    np.arange(64 + 16 * g, 64 + 16 * g + 16)])
    for g in range(4)])
RB = 1024
RB2 = 1000


def _dense_pre_body(x_ref, pos_ref, wcat_ref, wep_ref, be_ref,
                    base_ref, c_ref, xw1_ref):
    y = jnp.dot(x_ref[...], wcat_ref[...], preferred_element_type=jnp.float32)
    p2 = (pos_ref[:, 0:1] * wep_ref[0:1, :] + pos_ref[:, 1:2] * wep_ref[1:2, :])
    base_ref[...] = y[:, :O] - p2[:, :O] + be_ref[...]
    cf = y[:, O:2 * O] + p2[:, O:]
    cr = cf.astype(jnp.bfloat16).astype(jnp.float32)
    lo = lax.shift_right_logical(
        lax.bitcast_convert_type(cr[:, :O // 2], jnp.int32), 16)
    hi = lax.bitcast_convert_type(cr[:, O // 2:], jnp.int32) & jnp.int32(-65536)
    c_ref[...] = lo | hi
    xw1_ref[...] = y[:, 2 * O:]


def _node_body(xw1_ref, agg_ref, wn2_ref, bn_ref, lns_ref, lnb_ref, mask_ref,
               out_ref):
    h = (xw1_ref[...]
         + jnp.dot(agg_ref[...], wn2_ref[...], preferred_element_type=jnp.float32)
         + bn_ref[...])
    mu = jnp.mean(h, axis=1, keepdims=True)
    hc = h - mu
    var = jnp.mean(hc * hc, axis=1, keepdims=True)
    hn = hc * lax.rsqrt(var + 1e-5) * lns_ref[...] + lnb_ref[...]
    out_ref[...] = jnp.maximum(hn, 0.0) * mask_ref[...]


def _sc_gather_body(base_hbm, c_hbm, idx_hbm, out_hbm,
                    c_sp, idx_v, base_v, rows0, rows1, rows2, rows3,
                    sem0, sem1, sem2, sem3):
    cid = lax.axis_index("c")
    sid = lax.axis_index("s")
    wid = sid * NC + cid
    node0 = wid * NPT
    chunk0 = wid * NCHUNK

    rps = N_PAD // NS
    pltpu.sync_copy(c_hbm.at[pl.ds(sid * rps, rps)],
                    c_sp.at[pl.ds(sid * rps, rps)])

    pltpu.sync_copy(idx_hbm.at[pl.ds(chunk0, NCHUNK)], idx_v)
    pltpu.sync_copy(base_hbm.at[pl.ds(node0, NPT)], base_v)

    rows = (rows0, rows1, rows2, rows3)
    sems = (sem0, sem1, sem2, sem3)

    plsc.subcore_barrier()

    for b in range(NB):
        pltpu.async_copy(c_sp.at[idx_v.at[b]], rows[b], sems[b])

    inv_k = jnp.float32(1.0 / K)

    def ring_body(it, carry):
        for b in range(NB):
            g = it * NB + b
            pltpu.make_async_copy(c_sp.at[idx_v.at[g]], rows[b],
                                  sems[b]).wait()
            rb = rows[b]
            for i in range(CH):
                nrow = g * CH + i
                bjs = [base_v[nrow, pl.ds(L * j, L)] for j in range(8)]

                def kg_body(kg, accs):
                    r0 = i * K + kg * KU
                    new = list(accs)
                    hi_mask = jnp.full((L,), -65536, jnp.int32)
                    for kk in range(KU):
                        for jj in range(4):
                            vi = rb[r0 + kk, pl.ds(L * jj, L)]
                            ev = lax.bitcast_convert_type(vi << 16,
                                                          jnp.float32)
                            od = lax.bitcast_convert_type(vi & hi_mask,
                                                          jnp.float32)
                            new[2 * jj] = new[2 * jj] + jnp.maximum(
                                ev + bjs[2 * jj], 0.0)
                            new[2 * jj + 1] = new[2 * jj + 1] + jnp.maximum(
                                od + bjs[2 * jj + 1], 0.0)
                    return tuple(new)

                accs = lax.fori_loop(
                    0, K // KU, kg_body,
                    tuple(jnp.zeros((L,), jnp.float32) for _ in range(8)))
                for j in range(8):
                    base_v[nrow, pl.ds(L * j, L)] = accs[j] * inv_k
            @pl.when(g + NB < NCHUNK)
            def _():
                pltpu.async_copy(c_sp.at[idx_v.at[g + NB]], rows[b], sems[b])
        return carry

    lax.fori_loop(0, NCHUNK // NB, ring_body, jnp.int32(0))

    pltpu.sync_copy(base_v, out_hbm.at[pl.ds(node0, NPT)])


_sc_gather = functools.partial(
    pl.kernel,
    out_type=jax.ShapeDtypeStruct((N_PAD, O), jnp.float32),
    mesh=plsc.VectorSubcoreMesh(core_axis_name="c", subcore_axis_name="s",
                                num_cores=NC, num_subcores=NS),
    compiler_params=pltpu.CompilerParams(use_tc_tiling_on_sc=False),
    scratch_types=[
        pltpu.VMEM_SHARED((N_PAD, O // 2), jnp.int32),
        pltpu.VMEM((NCHUNK, RG), jnp.int32),
        pltpu.VMEM((NPT, O), jnp.float32),
        pltpu.VMEM((RG, O // 2), jnp.int32),
        pltpu.VMEM((RG, O // 2), jnp.int32),
        pltpu.VMEM((RG, O // 2), jnp.int32),
        pltpu.VMEM((RG, O // 2), jnp.int32),
        pltpu.SemaphoreType.DMA,
        pltpu.SemaphoreType.DMA,
        pltpu.SemaphoreType.DMA,
        pltpu.SemaphoreType.DMA,
    ],
)(_sc_gather_body)


def kernel(x, pos, edge_idx, mask, W_e, b_e, W_n, b_n, ln_scale, ln_bias):
    B = x.shape[0]
    x2 = x.reshape(B * N, D)
    pos2 = pos.reshape(B * N, 2)
    pad = N_PAD - N
    x_pad = jnp.pad(x2, ((0, pad), (0, 0)))
    pos_pad = jnp.pad(pos2, ((0, pad), (0, 0)))
    idx_pad = jnp.pad(edge_idx.reshape(B * N, K).astype(jnp.int32),
                      ((0, pad), (0, 0))).reshape(NW * NCHUNK, RG)

    perm = jnp.asarray(_PERM)
    wcat = jnp.concatenate([W_e[:D][:, perm], W_e[D:2 * D], W_n[:D]], axis=1)
    wep = jnp.concatenate([W_e[2 * D:][:, perm], W_e[2 * D:]], axis=1)
    wn2 = W_n[D:][perm, :]

    grid = N_PAD // RB
    base_c_xw1 = pl.pallas_call(
        _dense_pre_body,
        grid=(grid,),
        in_specs=[
            pl.BlockSpec((RB, D), lambda i: (i, 0)),
            pl.BlockSpec((RB, 2), lambda i: (i, 0)),
            pl.BlockSpec((D, 3 * O), lambda i: (0, 0)),
            pl.BlockSpec((2, 2 * O), lambda i: (0, 0)),
            pl.BlockSpec((1, O), lambda i: (0, 0)),
        ],
        out_specs=[
            pl.BlockSpec((RB, O), lambda i: (i, 0)),
            pl.BlockSpec((RB, O // 2), lambda i: (i, 0)),
            pl.BlockSpec((RB, O), lambda i: (i, 0)),
        ],
        out_shape=[
            jax.ShapeDtypeStruct((N_PAD, O), jnp.float32),
            jax.ShapeDtypeStruct((N_PAD, O // 2), jnp.int32),
            jax.ShapeDtypeStruct((N_PAD, O), jnp.float32),
        ],
    )(x_pad, pos_pad, wcat, wep, b_e[perm].reshape(1, O))
    base, c_i32, xw1 = base_c_xw1

    agg = _sc_gather(base, c_i32, idx_pad)

    out = pl.pallas_call(
        _node_body,
        grid=(N // RB2,),
        in_specs=[
            pl.BlockSpec((RB2, O), lambda i: (i, 0)),
            pl.BlockSpec((RB2, O), lambda i: (i, 0)),
            pl.BlockSpec((O, O), lambda i: (0, 0)),
            pl.BlockSpec((1, O), lambda i: (0, 0)),
            pl.BlockSpec((1, O), lambda i: (0, 0)),
            pl.BlockSpec((1, O), lambda i: (0, 0)),
            pl.BlockSpec((RB2, 1), lambda i: (i, 0)),
        ],
        out_specs=pl.BlockSpec((RB2, O), lambda i: (i, 0)),
        out_shape=jax.ShapeDtypeStruct((N, O), jnp.float32),
    )(xw1, agg, wn2, b_n.reshape(1, O), ln_scale.reshape(1, O),
      ln_bias.reshape(1, O), mask.reshape(B * N, 1))

    return out.reshape(B, N, O)


# speedup vs baseline: 3243.9633x; 1.3674x over previous
import functools

import numpy as np

import jax
import jax.numpy as jnp
from jax import lax
from jax.experimental import pallas as pl
from jax.experimental.pallas import tpu as pltpu
from jax.experimental.pallas import tpu_sc as plsc

N = 10000
K = 32
D = 128
O = 128

NC = 2
NS = 16
NW = NC * NS
L = 16
N_PAD = 10240
NPT = N_PAD // NW
CH = 2
RG = CH * K
NCHUNK = NPT // CH
NB = 4
KU = 8

RB = 1024
RB2 = 1000


def _dense_pre_body(x_ref, pos_ref, wcat_ref, wp_ref, be_ref,
                    base_ref, c_ref, xw1_ref):
    y = jnp.dot(x_ref[...], wcat_ref[...], preferred_element_type=jnp.float32)
    p2 = (pos_ref[:, 0:1] * wp_ref[0:1, :] + pos_ref[:, 1:2] * wp_ref[1:2, :])
    base_ref[...] = (y[:, :O] - p2 + be_ref[...]).astype(jnp.bfloat16)
    c_ref[...] = (y[:, O:2 * O] + p2).astype(jnp.bfloat16)
    xw1_ref[...] = y[:, 2 * O:]


def _node_body(xw1_ref, agg_ref, wn2_ref, bn_ref, lns_ref, lnb_ref, mask_ref,
               out_ref):
    h = (xw1_ref[...]
         + jnp.dot(agg_ref[...].astype(jnp.float32), wn2_ref[...],
                   preferred_element_type=jnp.float32)
         + bn_ref[...])
    mu = jnp.mean(h, axis=1, keepdims=True)
    hc = h - mu
    var = jnp.mean(hc * hc, axis=1, keepdims=True)
    hn = hc * lax.rsqrt(var + 1e-5) * lns_ref[...] + lnb_ref[...]
    out_ref[...] = jnp.maximum(hn, 0.0) * mask_ref[...]


def _sc_gather_body(base_hbm, c_hbm, idx_hbm, out_hbm,
                    c_sp, idx_v, base_v, out_v, rows0, rows1, rows2, rows3,
                    sem0, sem1, sem2, sem3):
    cid = lax.axis_index("c")
    sid = lax.axis_index("s")
    wid = sid * NC + cid
    node0 = wid * NPT
    chunk0 = wid * NCHUNK

    rps = N_PAD // NS
    pltpu.sync_copy(c_hbm.at[pl.ds(sid * rps, rps)],
                    c_sp.at[pl.ds(sid * rps, rps)])

    pltpu.sync_copy(idx_hbm.at[pl.ds(chunk0, NCHUNK)], idx_v)
    pltpu.sync_copy(base_hbm.at[pl.ds(node0, NPT)], base_v)

    rows = (rows0, rows1, rows2, rows3)
    sems = (sem0, sem1, sem2, sem3)

    plsc.subcore_barrier()

    for b in range(NB):
        pltpu.async_copy(c_sp.at[idx_v.at[b]], rows[b], sems[b])

    W = 2 * L
    zero_bf = jnp.zeros((W,), jnp.bfloat16)

    def ring_body(it, carry):
        for b in range(NB):
            g = it * NB + b
            pltpu.make_async_copy(c_sp.at[idx_v.at[g]], rows[b],
                                  sems[b]).wait()
            rb = rows[b]
            for i in range(CH):
                nrow = g * CH + i
                bjs = [base_v[nrow, pl.ds(W * q, W)] for q in range(4)]

                def kg_body(kg, accs):
                    r0 = i * K + kg * KU
                    new = list(accs)
                    for q in range(4):
                        ms = [jnp.maximum(rb[r0 + kk, pl.ds(W * q, W)]
                                          + bjs[q], zero_bf)
                              for kk in range(KU)]
                        while len(ms) > 1:
                            ms = [ms[z] + ms[z + 1]
                                  for z in range(0, len(ms), 2)]
                        new[q] = new[q] + ms[0]
                    return tuple(new)

                accs = lax.fori_loop(
                    0, K // KU, kg_body,
                    tuple(jnp.zeros((W,), jnp.bfloat16) for _ in range(4)))
                for q in range(4):
                    out_v[nrow, pl.ds(W * q, W)] = accs[q]
            @pl.when(g + NB < NCHUNK)
            def _():
                pltpu.async_copy(c_sp.at[idx_v.at[g + NB]], rows[b], sems[b])
        return carry

    lax.fori_loop(0, NCHUNK // NB, ring_body, jnp.int32(0))

    pltpu.sync_copy(out_v, out_hbm.at[pl.ds(node0, NPT)])


_sc_gather = functools.partial(
    pl.kernel,
    out_type=jax.ShapeDtypeStruct((N_PAD, O), jnp.bfloat16),
    mesh=plsc.VectorSubcoreMesh(core_axis_name="c", subcore_axis_name="s",
                                num_cores=NC, num_subcores=NS),
    compiler_params=pltpu.CompilerParams(use_tc_tiling_on_sc=False),
    scratch_types=[
        pltpu.VMEM_SHARED((N_PAD, O), jnp.bfloat16),
        pltpu.VMEM((NCHUNK, RG), jnp.int32),
        pltpu.VMEM((NPT, O), jnp.bfloat16),
        pltpu.VMEM((NPT, O), jnp.bfloat16),
        pltpu.VMEM((RG, O), jnp.bfloat16),
        pltpu.VMEM((RG, O), jnp.bfloat16),
        pltpu.VMEM((RG, O), jnp.bfloat16),
        pltpu.VMEM((RG, O), jnp.bfloat16),
        pltpu.SemaphoreType.DMA,
        pltpu.SemaphoreType.DMA,
        pltpu.SemaphoreType.DMA,
        pltpu.SemaphoreType.DMA,
    ],
)(_sc_gather_body)


def kernel(x, pos, edge_idx, mask, W_e, b_e, W_n, b_n, ln_scale, ln_bias):
    B = x.shape[0]
    x2 = x.reshape(B * N, D)
    pos2 = pos.reshape(B * N, 2)
    pad = N_PAD - N
    x_pad = jnp.pad(x2, ((0, pad), (0, 0)))
    pos_pad = jnp.pad(pos2, ((0, pad), (0, 0)))
    idx_pad = jnp.pad(edge_idx.reshape(B * N, K).astype(jnp.int32),
                      ((0, pad), (0, 0))).reshape(NW * NCHUNK, RG)

    wcat = jnp.concatenate([W_e[:D], W_e[D:2 * D], W_n[:D]], axis=1)
    wn2 = W_n[D:] * jnp.float32(1.0 / K)

    grid = N_PAD // RB
    base_c_xw1 = pl.pallas_call(
        _dense_pre_body,
        grid=(grid,),
        in_specs=[
            pl.BlockSpec((RB, D), lambda i: (i, 0)),
            pl.BlockSpec((RB, 2), lambda i: (i, 0)),
            pl.BlockSpec((D, 3 * O), lambda i: (0, 0)),
            pl.BlockSpec((2, O), lambda i: (0, 0)),
            pl.BlockSpec((1, O), lambda i: (0, 0)),
        ],
        out_specs=[
            pl.BlockSpec((RB, O), lambda i: (i, 0)),
            pl.BlockSpec((RB, O), lambda i: (i, 0)),
            pl.BlockSpec((RB, O), lambda i: (i, 0)),
        ],
        out_shape=[
            jax.ShapeDtypeStruct((N_PAD, O), jnp.bfloat16),
            jax.ShapeDtypeStruct((N_PAD, O), jnp.bfloat16),
            jax.ShapeDtypeStruct((N_PAD, O), jnp.float32),
        ],
    )(x_pad, pos_pad, wcat, W_e[2 * D:], b_e.reshape(1, O))
    base, c_i32, xw1 = base_c_xw1

    agg = _sc_gather(base, c_i32, idx_pad)

    out = pl.pallas_call(
        _node_body,
        grid=(N // RB2,),
        in_specs=[
            pl.BlockSpec((RB2, O), lambda i: (i, 0)),
            pl.BlockSpec((RB2, O), lambda i: (i, 0)),
            pl.BlockSpec((O, O), lambda i: (0, 0)),
            pl.BlockSpec((1, O), lambda i: (0, 0)),
            pl.BlockSpec((1, O), lambda i: (0, 0)),
            pl.BlockSpec((1, O), lambda i: (0, 0)),
            pl.BlockSpec((RB2, 1), lambda i: (i, 0)),
        ],
        out_specs=pl.BlockSpec((RB2, O), lambda i: (i, 0)),
        out_shape=jax.ShapeDtypeStruct((N, O), jnp.float32),
    )(xw1, agg, wn2, b_n.reshape(1, O), ln_scale.reshape(1, O),
      ln_bias.reshape(1, O), mask.reshape(B * N, 1))

    return out.reshape(B, N, O)


# speedup vs baseline: 3413.9982x; 1.0524x over previous
import functools

import numpy as np

import jax
import jax.numpy as jnp
from jax import lax
from jax.experimental import pallas as pl
from jax.experimental.pallas import tpu as pltpu
from jax.experimental.pallas import tpu_sc as plsc

N = 10000
K = 32
D = 128
O = 128

NC = 2
NS = 16
NW = NC * NS
L = 16
N_PAD = 10240
NPT = N_PAD // NW
CH = 2
RG = CH * K
NCHUNK = NPT // CH
NB = 4
KU = 16

RB = 1024
RB2 = 1000


def _dense_pre_body(x_ref, pos_ref, wcat_ref, wp_ref, be_ref,
                    base_ref, c_ref, xw1_ref):
    y = jnp.dot(x_ref[...], wcat_ref[...], preferred_element_type=jnp.float32)
    p2 = (pos_ref[:, 0:1] * wp_ref[0:1, :] + pos_ref[:, 1:2] * wp_ref[1:2, :])
    base_ref[...] = (y[:, :O] - p2 + be_ref[...]).astype(jnp.bfloat16)
    c_ref[...] = (y[:, O:2 * O] + p2).astype(jnp.bfloat16)
    xw1_ref[...] = y[:, 2 * O:]


def _node_body(xw1_ref, agg_ref, wn2_ref, bn_ref, lns_ref, lnb_ref, mask_ref,
               out_ref):
    h = (xw1_ref[...]
         + jnp.dot(agg_ref[...].astype(jnp.float32), wn2_ref[...],
                   preferred_element_type=jnp.float32)
         + bn_ref[...])
    mu = jnp.mean(h, axis=1, keepdims=True)
    hc = h - mu
    var = jnp.mean(hc * hc, axis=1, keepdims=True)
    hn = hc * lax.rsqrt(var + 1e-5) * lns_ref[...] + lnb_ref[...]
    out_ref[...] = jnp.maximum(hn, 0.0) * mask_ref[...]


def _sc_gather_body(base_hbm, c_hbm, idx_hbm, out_hbm,
                    c_sp, idx_v, base_v, out_v, rows0, rows1, rows2, rows3,
                    sem0, sem1, sem2, sem3):
    cid = lax.axis_index("c")
    sid = lax.axis_index("s")
    wid = sid * NC + cid
    node0 = wid * NPT
    chunk0 = wid * NCHUNK

    rps = N_PAD // NS
    pltpu.sync_copy(c_hbm.at[pl.ds(sid * rps, rps)],
                    c_sp.at[pl.ds(sid * rps, rps)])

    pltpu.sync_copy(idx_hbm.at[pl.ds(chunk0, NCHUNK)], idx_v)
    pltpu.sync_copy(base_hbm.at[pl.ds(node0, NPT)], base_v)

    rows = (rows0, rows1, rows2, rows3)
    sems = (sem0, sem1, sem2, sem3)

    plsc.subcore_barrier()

    for b in range(NB):
        pltpu.async_copy(c_sp.at[idx_v.at[b]], rows[b], sems[b])

    W = 2 * L
    zero_bf = jnp.zeros((W,), jnp.bfloat16)

    def ring_body(it, carry):
        for b in range(NB):
            g = it * NB + b
            pltpu.make_async_copy(c_sp.at[idx_v.at[g]], rows[b],
                                  sems[b]).wait()
            rb = rows[b]
            for i in range(CH):
                nrow = g * CH + i
                bjs = [base_v[nrow, pl.ds(W * q, W)] for q in range(4)]

                def kg_body(kg, accs):
                    r0 = i * K + kg * KU
                    new = list(accs)
                    for q in range(4):
                        ms = [jnp.maximum(rb[r0 + kk, pl.ds(W * q, W)]
                                          + bjs[q], zero_bf)
                              for kk in range(KU)]
                        while len(ms) > 1:
                            ms = [ms[z] + ms[z + 1]
                                  for z in range(0, len(ms), 2)]
                        new[q] = new[q] + ms[0]
                    return tuple(new)

                accs = lax.fori_loop(
                    0, K // KU, kg_body,
                    tuple(jnp.zeros((W,), jnp.bfloat16) for _ in range(4)))
                for q in range(4):
                    out_v[nrow, pl.ds(W * q, W)] = accs[q]
            @pl.when(g + NB < NCHUNK)
            def _():
                pltpu.async_copy(c_sp.at[idx_v.at[g + NB]], rows[b], sems[b])
        return carry

    lax.fori_loop(0, NCHUNK // NB, ring_body, jnp.int32(0))

    pltpu.sync_copy(out_v, out_hbm.at[pl.ds(node0, NPT)])


_sc_gather = functools.partial(
    pl.kernel,
    out_type=jax.ShapeDtypeStruct((N_PAD, O), jnp.bfloat16),
    mesh=plsc.VectorSubcoreMesh(core_axis_name="c", subcore_axis_name="s",
                                num_cores=NC, num_subcores=NS),
    compiler_params=pltpu.CompilerParams(use_tc_tiling_on_sc=False),
    scratch_types=[
        pltpu.VMEM_SHARED((N_PAD, O), jnp.bfloat16),
        pltpu.VMEM((NCHUNK, RG), jnp.int32),
        pltpu.VMEM((NPT, O), jnp.bfloat16),
        pltpu.VMEM((NPT, O), jnp.bfloat16),
        pltpu.VMEM((RG, O), jnp.bfloat16),
        pltpu.VMEM((RG, O), jnp.bfloat16),
        pltpu.VMEM((RG, O), jnp.bfloat16),
        pltpu.VMEM((RG, O), jnp.bfloat16),
        pltpu.SemaphoreType.DMA,
        pltpu.SemaphoreType.DMA,
        pltpu.SemaphoreType.DMA,
        pltpu.SemaphoreType.DMA,
    ],
)(_sc_gather_body)


def kernel(x, pos, edge_idx, mask, W_e, b_e, W_n, b_n, ln_scale, ln_bias):
    B = x.shape[0]
    x2 = x.reshape(B * N, D)
    pos2 = pos.reshape(B * N, 2)
    pad = N_PAD - N
    idx_pad = jnp.pad(edge_idx.reshape(B * N, K).astype(jnp.int32),
                      ((0, pad), (0, 0))).reshape(NW * NCHUNK, RG)

    wcat = jnp.concatenate([W_e[:D], W_e[D:2 * D], W_n[:D]], axis=1)
    wn2 = W_n[D:] * jnp.float32(1.0 / K)

    grid = N_PAD // RB
    base_c_xw1 = pl.pallas_call(
        _dense_pre_body,
        grid=(grid,),
        in_specs=[
            pl.BlockSpec((RB, D), lambda i: (i, 0)),
            pl.BlockSpec((RB, 2), lambda i: (i, 0)),
            pl.BlockSpec((D, 3 * O), lambda i: (0, 0)),
            pl.BlockSpec((2, O), lambda i: (0, 0)),
            pl.BlockSpec((1, O), lambda i: (0, 0)),
        ],
        out_specs=[
            pl.BlockSpec((RB, O), lambda i: (i, 0)),
            pl.BlockSpec((RB, O), lambda i: (i, 0)),
            pl.BlockSpec((RB, O), lambda i: (i, 0)),
        ],
        out_shape=[
            jax.ShapeDtypeStruct((N_PAD, O), jnp.bfloat16),
            jax.ShapeDtypeStruct((N_PAD, O), jnp.bfloat16),
            jax.ShapeDtypeStruct((N_PAD, O), jnp.float32),
        ],
    )(x2, pos2, wcat, W_e[2 * D:], b_e.reshape(1, O))
    base, c_i32, xw1 = base_c_xw1

    agg = _sc_gather(base, c_i32, idx_pad)

    out = pl.pallas_call(
        _node_body,
        grid=(N // RB2,),
        in_specs=[
            pl.BlockSpec((RB2, O), lambda i: (i, 0)),
            pl.BlockSpec((RB2, O), lambda i: (i, 0)),
            pl.BlockSpec((O, O), lambda i: (0, 0)),
            pl.BlockSpec((1, O), lambda i: (0, 0)),
            pl.BlockSpec((1, O), lambda i: (0, 0)),
            pl.BlockSpec((1, O), lambda i: (0, 0)),
            pl.BlockSpec((RB2, 1), lambda i: (i, 0)),
        ],
        out_specs=pl.BlockSpec((RB2, O), lambda i: (i, 0)),
        out_shape=jax.ShapeDtypeStruct((N, O), jnp.float32),
    )(xw1, agg, wn2, b_n.reshape(1, O), ln_scale.reshape(1, O),
      ln_bias.reshape(1, O), mask.reshape(B * N, 1))

    return out.reshape(B, N, O)
